# baseline (speedup 1.0000x reference)
import functools

import jax
import jax.numpy as jnp
from jax import lax
from jax.experimental import pallas as pl
from jax.experimental.pallas import tpu as pltpu
from jax.experimental.pallas import tpu_sc as plsc

VOCAB = 1000000
EMBED_DIM = 32
BATCH = 16384
HIST = 50

_NUM_CORES = 2
_NUM_SUBCORES = 16
_NW = _NUM_CORES * _NUM_SUBCORES
_BPW = BATCH // _NW
_NBT = _BPW // 128


def _gather_kernel(x_hbm, table_hbm, out_hbm, xb, xT, rows0, rows1, o0, o1,
                   sg0, sg1, ss0, ss1):
    wid = lax.axis_index("s") * _NUM_CORES + lax.axis_index("c")
    b0w = wid * _BPW
    lane = lax.iota(jnp.int32, 16)

    pltpu.sync_copy(x_hbm.at[pl.ds(b0w, _BPW)], xb)

    def tr_x_h(h, carry):
        col16 = jnp.full((16,), h, jnp.int32)

        def tr_x_b(g, c2):
            row16 = g * 16 + lane
            vals = plsc.load_gather(xb, [row16, col16])
            xT[pl.ds(h * _BPW + g * 16, 16)] = vals
            return c2

        return lax.fori_loop(0, _BPW // 16, tr_x_b, carry)

    lax.fori_loop(0, HIST, tr_x_h, 0)

    def start_gather(h, rows_v, sem):
        pltpu.async_copy(
            table_hbm.at[xT.at[pl.ds(h * _BPW, _BPW)]], rows_v, sem)

    def wait_gather(h, rows_v, sem):
        pltpu.make_async_copy(
            table_hbm.at[xT.at[pl.ds(h * _BPW, _BPW)]], rows_v, sem).wait()

    def transpose_block(rows_v, o_v):
        def tb(t, carry):
            dt = t // _NBT
            bti = t % _NBT
            colbase = dt * 8

            def tg(g, c2):
                row16 = bti * 128 + g * 16 + lane
                for r in range(8):
                    col16 = jnp.full((16,), colbase + r, jnp.int32)
                    vals = plsc.load_gather(rows_v, [row16, col16])
                    o_v[t, r, pl.ds(g * 16, 16)] = vals
                return c2

            return lax.fori_loop(0, 8, tg, carry)

        lax.fori_loop(0, 4 * _NBT, tb, 0)

    def start_stores(h, o_v, sem):
        def st(t, carry):
            dt = t // _NBT
            bti = t % _NBT
            pltpu.async_copy(
                o_v.at[t], out_hbm.at[h, dt, _NBT * wid + bti], sem)
            return carry

        lax.fori_loop(0, 4 * _NBT, st, 0)

    def wait_stores(h, o_v, sem):
        def st(t, carry):
            dt = t // _NBT
            bti = t % _NBT
            pltpu.make_async_copy(
                o_v.at[t], out_hbm.at[h, dt, _NBT * wid + bti], sem).wait()
            return carry

        lax.fori_loop(0, 4 * _NBT, st, 0)

    start_gather(0, rows0, sg0)

    def pair_body(g, carry):
        h0 = 2 * g
        h1 = h0 + 1

        start_gather(h1, rows1, sg1)
        wait_gather(h0, rows0, sg0)

        @pl.when(g > 0)
        def _():
            wait_stores(h0 - 2, o0, ss0)

        transpose_block(rows0, o0)
        start_stores(h0, o0, ss0)

        @pl.when(h1 + 1 < HIST)
        def _():
            start_gather(h0 + 2, rows0, sg0)

        wait_gather(h1, rows1, sg1)

        @pl.when(g > 0)
        def _():
            wait_stores(h1 - 2, o1, ss1)

        transpose_block(rows1, o1)
        start_stores(h1, o1, ss1)
        return carry

    lax.fori_loop(0, HIST // 2, pair_body, 0, unroll=False)

    wait_stores(HIST - 2, o0, ss0)
    wait_stores(HIST - 1, o1, ss1)


@jax.jit
def _embedding_gather(x, table):
    mesh = plsc.VectorSubcoreMesh(core_axis_name="c", subcore_axis_name="s")
    k = functools.partial(
        pl.kernel,
        mesh=mesh,
        out_type=jax.ShapeDtypeStruct(
            (HIST, EMBED_DIM // 8, BATCH // 128, 8, 128), jnp.float32),
        scratch_types=[
            pltpu.VMEM((_BPW, HIST), jnp.int32),
            pltpu.VMEM((_BPW * HIST,), jnp.int32),
            pltpu.VMEM((_BPW, EMBED_DIM), jnp.float32),
            pltpu.VMEM((_BPW, EMBED_DIM), jnp.float32),
            pltpu.VMEM((4 * _NBT, 8, 128), jnp.float32),
            pltpu.VMEM((4 * _NBT, 8, 128), jnp.float32),
            pltpu.SemaphoreType.DMA,
            pltpu.SemaphoreType.DMA,
            pltpu.SemaphoreType.DMA,
            pltpu.SemaphoreType.DMA,
        ],
        compiler_params=pltpu.CompilerParams(
            use_tc_tiling_on_sc=False, needs_layout_passes=False),
    )(_gather_kernel)
    return k(x, table)


def kernel(x, table):
    p = _embedding_gather(x, table)
    return p.transpose(2, 4, 0, 1, 3).reshape(BATCH, HIST, EMBED_DIM)


# speedup vs baseline: 1.0085x; 1.0085x over previous
import functools

import jax
import jax.numpy as jnp
from jax import lax
from jax.experimental import pallas as pl
from jax.experimental.pallas import tpu as pltpu
from jax.experimental.pallas import tpu_sc as plsc

VOCAB = 1000000
EMBED_DIM = 32
BATCH = 16384
HIST = 50

_NUM_CORES = 2
_NUM_SUBCORES = 16
_NW = _NUM_CORES * _NUM_SUBCORES
_BPW = BATCH // _NW
_NBT = _BPW // 128


def _gather_kernel(x_hbm, table_hbm, out_hbm, xb, xT, rows0, rows1, o0, o1,
                   sg0, sg1, ss0, ss1):
    wid = lax.axis_index("s") * _NUM_CORES + lax.axis_index("c")
    b0w = wid * _BPW
    lane = lax.iota(jnp.int32, 16)

    pltpu.sync_copy(x_hbm.at[pl.ds(b0w, _BPW)], xb)

    def tr_x_h(h, carry):
        col16 = jnp.full((16,), h, jnp.int32)

        def tr_x_b(g, c2):
            row16 = g * 16 + lane
            vals = plsc.load_gather(xb, [row16, col16])
            xT[pl.ds(h * _BPW + g * 16, 16)] = vals
            return c2

        return lax.fori_loop(0, _BPW // 16, tr_x_b, carry)

    lax.fori_loop(0, HIST, tr_x_h, 0)

    def start_gather(h, rows_v, sem):
        pltpu.async_copy(
            table_hbm.at[xT.at[pl.ds(h * _BPW, _BPW)]], rows_v, sem)

    def wait_gather(h, rows_v, sem):
        pltpu.make_async_copy(
            table_hbm.at[xT.at[pl.ds(h * _BPW, _BPW)]], rows_v, sem).wait()

    def transpose_block(rows_v, o_v):
        def tb(t, carry):
            dt = t // _NBT
            bti = t % _NBT
            colbase = dt * 8
            rowbase = bti * 128 + lane
            rows16 = [rowbase + g * 16 for g in range(8)]
            cols16 = [jnp.full((16,), colbase + r, jnp.int32)
                      for r in range(8)]
            for r in range(8):
                for g in range(8):
                    vals = plsc.load_gather(rows_v, [rows16[g], cols16[r]])
                    o_v[t, r, pl.ds(g * 16, 16)] = vals
            return carry

        lax.fori_loop(0, 4 * _NBT, tb, 0)

    def start_stores(h, o_v, sem):
        def st(t, carry):
            dt = t // _NBT
            bti = t % _NBT
            pltpu.async_copy(
                o_v.at[t], out_hbm.at[h, dt, _NBT * wid + bti], sem)
            return carry

        lax.fori_loop(0, 4 * _NBT, st, 0)

    def wait_stores(h, o_v, sem):
        def st(t, carry):
            dt = t // _NBT
            bti = t % _NBT
            pltpu.make_async_copy(
                o_v.at[t], out_hbm.at[h, dt, _NBT * wid + bti], sem).wait()
            return carry

        lax.fori_loop(0, 4 * _NBT, st, 0)

    start_gather(0, rows0, sg0)

    def pair_body(g, carry):
        h0 = 2 * g
        h1 = h0 + 1

        start_gather(h1, rows1, sg1)
        wait_gather(h0, rows0, sg0)

        @pl.when(g > 0)
        def _():
            wait_stores(h0 - 2, o0, ss0)

        transpose_block(rows0, o0)
        start_stores(h0, o0, ss0)

        @pl.when(h1 + 1 < HIST)
        def _():
            start_gather(h0 + 2, rows0, sg0)

        wait_gather(h1, rows1, sg1)

        @pl.when(g > 0)
        def _():
            wait_stores(h1 - 2, o1, ss1)

        transpose_block(rows1, o1)
        start_stores(h1, o1, ss1)
        return carry

    lax.fori_loop(0, HIST // 2, pair_body, 0, unroll=False)

    wait_stores(HIST - 2, o0, ss0)
    wait_stores(HIST - 1, o1, ss1)


@jax.jit
def _embedding_gather(x, table):
    mesh = plsc.VectorSubcoreMesh(core_axis_name="c", subcore_axis_name="s")
    k = functools.partial(
        pl.kernel,
        mesh=mesh,
        out_type=jax.ShapeDtypeStruct(
            (HIST, EMBED_DIM // 8, BATCH // 128, 8, 128), jnp.float32),
        scratch_types=[
            pltpu.VMEM((_BPW, HIST), jnp.int32),
            pltpu.VMEM((_BPW * HIST,), jnp.int32),
            pltpu.VMEM((_BPW, EMBED_DIM), jnp.float32),
            pltpu.VMEM((_BPW, EMBED_DIM), jnp.float32),
            pltpu.VMEM((4 * _NBT, 8, 128), jnp.float32),
            pltpu.VMEM((4 * _NBT, 8, 128), jnp.float32),
            pltpu.SemaphoreType.DMA,
            pltpu.SemaphoreType.DMA,
            pltpu.SemaphoreType.DMA,
            pltpu.SemaphoreType.DMA,
        ],
        compiler_params=pltpu.CompilerParams(
            use_tc_tiling_on_sc=False, needs_layout_passes=False),
    )(_gather_kernel)
    return k(x, table)


def kernel(x, table):
    p = _embedding_gather(x, table)
    return p.transpose(2, 4, 0, 1, 3).reshape(BATCH, HIST, EMBED_DIM)


# speedup vs baseline: 1.1192x; 1.1098x over previous
import functools

import jax
import jax.numpy as jnp
from jax import lax
from jax.experimental import pallas as pl
from jax.experimental.pallas import tpu as pltpu
from jax.experimental.pallas import tpu_sc as plsc

VOCAB = 1000000
EMBED_DIM = 32
BATCH = 16384
HIST = 50

_NUM_CORES = 2
_NUM_SUBCORES = 16
_NW = _NUM_CORES * _NUM_SUBCORES

_B = BATCH * HIST
_BPW = BATCH // _NW
_NB = 16
_CIDX = _NB * HIST
_NCHUNK = _BPW // _NB


def _gather_kernel(idx_hbm, table_hbm, out_hbm, idx0, idx1, rows0, rows1,
                   si0, si1, sg0, sg1, ss0, ss1):
    wid = lax.axis_index("s") * _NUM_CORES + lax.axis_index("c")
    base_b = wid * _BPW

    def idx_src(i):
        return idx_hbm.at[pl.ds(base_b + i * _NB, _NB)]

    def out_dst(i):
        return out_hbm.at[pl.ds(base_b + i * _NB, _NB)]

    def start_gathers(idx_v, rows_v, sem):
        for j in range(_NB):
            pltpu.async_copy(
                table_hbm.at[idx_v.at[j]], rows_v.at[j], sem)

    def wait_gathers(idx_v, rows_v, sem):
        for j in range(_NB):
            pltpu.make_async_copy(
                table_hbm.at[idx_v.at[j]], rows_v.at[j], sem).wait()

    pltpu.async_copy(idx_src(0), idx0, si0)
    pltpu.async_copy(idx_src(1), idx1, si1)

    pltpu.make_async_copy(idx_src(0), idx0, si0).wait()
    start_gathers(idx0, rows0, sg0)

    pltpu.make_async_copy(idx_src(1), idx1, si1).wait()
    start_gathers(idx1, rows1, sg1)
    wait_gathers(idx0, rows0, sg0)
    pltpu.async_copy(rows0, out_dst(0), ss0)
    pltpu.async_copy(idx_src(2), idx0, si0)

    def pair_body(g, carry):
        i0 = 2 * g
        i1 = i0 + 1

        pltpu.make_async_copy(idx_src(i0), idx0, si0).wait()
        pltpu.make_async_copy(rows0, out_dst(i0 - 2), ss0).wait()
        start_gathers(idx0, rows0, sg0)

        wait_gathers(idx1, rows1, sg1)
        pltpu.async_copy(rows1, out_dst(i0 - 1), ss1)
        pltpu.async_copy(idx_src(i1), idx1, si1)

        pltpu.make_async_copy(idx_src(i1), idx1, si1).wait()
        pltpu.make_async_copy(rows1, out_dst(i0 - 1), ss1).wait()
        start_gathers(idx1, rows1, sg1)

        wait_gathers(idx0, rows0, sg0)
        pltpu.async_copy(rows0, out_dst(i0), ss0)

        @pl.when(i1 + 1 < _NCHUNK)
        def _():
            pltpu.async_copy(idx_src(i1 + 1), idx0, si0)

        return carry

    lax.fori_loop(1, _NCHUNK // 2, pair_body, 0, unroll=False)

    last = _NCHUNK - 1
    wait_gathers(idx1, rows1, sg1)
    pltpu.async_copy(rows1, out_dst(last), ss1)
    pltpu.make_async_copy(rows0, out_dst(last - 1), ss0).wait()
    pltpu.make_async_copy(rows1, out_dst(last), ss1).wait()


@jax.jit
def _embedding_gather(idx2d, table):
    mesh = plsc.VectorSubcoreMesh(core_axis_name="c", subcore_axis_name="s")
    k = functools.partial(
        pl.kernel,
        mesh=mesh,
        out_type=jax.ShapeDtypeStruct((BATCH, HIST, EMBED_DIM), jnp.float32),
        scratch_types=[
            pltpu.VMEM((_NB, HIST), jnp.int32),
            pltpu.VMEM((_NB, HIST), jnp.int32),
            pltpu.VMEM((_NB, HIST, EMBED_DIM), jnp.float32),
            pltpu.VMEM((_NB, HIST, EMBED_DIM), jnp.float32),
            pltpu.SemaphoreType.DMA,
            pltpu.SemaphoreType.DMA,
            pltpu.SemaphoreType.DMA,
            pltpu.SemaphoreType.DMA,
            pltpu.SemaphoreType.DMA,
            pltpu.SemaphoreType.DMA,
        ],
        compiler_params=pltpu.CompilerParams(use_tc_tiling_on_sc=False),
    )(_gather_kernel)
    return k(idx2d, table)


def kernel(x, table):
    return _embedding_gather(x, table)
